# baseline (speedup 1.0000x reference)
import jax
import jax.numpy as jnp
from jax.experimental import pallas as pl
from jax.experimental.pallas import tpu as pltpu

_XS = 2
_BMS = 2048


def _mlp_fused(x0, x1, w1_ref, b1_ref, wpt_ref, bp_ref, out_ref):
    bpt = jnp.transpose(bp_ref[...])
    for k, xr in enumerate((x0, x1)):
        feats = jnp.tanh(
            jnp.dot(xr[...], w1_ref[...], preferred_element_type=jnp.float32)
            + b1_ref[...]
        )
        out_t = jax.lax.dot_general(
            wpt_ref[...], feats,
            (((1,), (1,)), ((), ())),
            preferred_element_type=jnp.float32,
        )
        out_ref[:, pl.ds(k * _BMS, _BMS)] = out_t + bpt


def kernel(inputs, W1, b1, W2, b2, Wp, bp):
    B, D = inputs.shape
    H = W1.shape[1]
    O = Wp.shape[1]
    bm = _XS * _BMS
    b1r = b1.reshape(1, H)
    wpt = Wp.T
    bpr = bp.reshape(1, O)

    def xspec(k):
        return pl.BlockSpec((_BMS, D), lambda i, k=k: (_XS * i + k, 0))

    out_t = pl.pallas_call(
        _mlp_fused,
        grid=(B // bm,),
        in_specs=[
            xspec(0), xspec(1),
            pl.BlockSpec((D, H), lambda i: (0, 0)),
            pl.BlockSpec((1, H), lambda i: (0, 0)),
            pl.BlockSpec((O, D), lambda i: (0, 0)),
            pl.BlockSpec((1, O), lambda i: (0, 0)),
        ],
        out_specs=pl.BlockSpec((O, bm), lambda i: (0, i)),
        out_shape=jax.ShapeDtypeStruct((O, B), jnp.float32),
        compiler_params=pltpu.CompilerParams(
            dimension_semantics=("parallel",),
        ),
    )(inputs, inputs, W1, b1r, wpt, bpr)
    return out_t.T


# speedup vs baseline: 1.0095x; 1.0095x over previous
import jax
import jax.numpy as jnp
from jax.experimental import pallas as pl
from jax.experimental.pallas import tpu as pltpu

_XS = 2
_BMS = 2048


def _mlp_fused(x0, x1, w1_ref, b1_ref, wpt_ref, bp_ref, out_ref):
    bpt = jnp.transpose(bp_ref[...])
    for k, xr in enumerate((x0, x1)):
        feats = jnp.tanh(
            jnp.dot(xr[...].astype(jnp.bfloat16), w1_ref[...].astype(jnp.bfloat16),
                    preferred_element_type=jnp.float32)
            + b1_ref[...]
        )
        out_t = jax.lax.dot_general(
            wpt_ref[...], feats,
            (((1,), (1,)), ((), ())),
            preferred_element_type=jnp.float32,
        )
        out_ref[:, pl.ds(k * _BMS, _BMS)] = out_t + bpt


def kernel(inputs, W1, b1, W2, b2, Wp, bp):
    B, D = inputs.shape
    H = W1.shape[1]
    O = Wp.shape[1]
    bm = _XS * _BMS
    b1r = b1.reshape(1, H)
    wpt = Wp.T
    bpr = bp.reshape(1, O)

    def xspec(k):
        return pl.BlockSpec((_BMS, D), lambda i, k=k: (_XS * i + k, 0))

    out_t = pl.pallas_call(
        _mlp_fused,
        grid=(B // bm,),
        in_specs=[
            xspec(0), xspec(1),
            pl.BlockSpec((D, H), lambda i: (0, 0)),
            pl.BlockSpec((1, H), lambda i: (0, 0)),
            pl.BlockSpec((O, D), lambda i: (0, 0)),
            pl.BlockSpec((1, O), lambda i: (0, 0)),
        ],
        out_specs=pl.BlockSpec((O, bm), lambda i: (0, i)),
        out_shape=jax.ShapeDtypeStruct((O, B), jnp.float32),
        compiler_params=pltpu.CompilerParams(
            dimension_semantics=("parallel",),
        ),
    )(inputs, inputs, W1, b1r, wpt, bpr)
    return out_t.T
